# Initial kernel scaffold; baseline (speedup 1.0000x reference)
import jax
import jax.numpy as jnp
from jax.experimental import pallas as pl


def kernel(input_sequence, embedding_table, f):
    raise NotImplementedError("write your pallas kernel here")


# speedup vs baseline: 4.8391x; 4.8391x over previous
import functools

import jax
import jax.numpy as jnp
from jax import lax
from jax.experimental import pallas as pl
from jax.experimental.pallas import tpu as pltpu
from jax.experimental.pallas import tpu_sc as plsc

EMBED = 32
SEQ = 20
HALF = 16

NUM_WORKERS = 32
TOKENS = 4096 * 26
TOK_PER_W = TOKENS // NUM_WORKERS
CHUNK_TOK = 64
CHUNKS = TOK_PER_W // CHUNK_TOK
ROWS_PER_CHUNK = CHUNK_TOK * SEQ
IDX_PER_GATHER = 128
GATHERS = ROWS_PER_CHUNK // IDX_PER_GATHER
IDX_ROWS_PER_CHUNK = GATHERS
IDX_ROWS_PER_W = CHUNKS * IDX_ROWS_PER_CHUNK


def _body(table, idx, f, out, idx_v, rows_v, f_v, out_v, gsem):
    wid = lax.axis_index("s") * 2 + lax.axis_index("c")
    pltpu.sync_copy(f, f_v)

    def chunk_body(c, carry):
        i0 = (wid * TOK_PER_W + c * CHUNK_TOK) * SEQ
        pltpu.sync_copy(idx.at[pl.ds(i0, ROWS_PER_CHUNK)], idx_v)
        copies = [
            pltpu.async_copy(
                table.at[idx_v.at[pl.ds(j * IDX_PER_GATHER, IDX_PER_GATHER)]],
                rows_v.at[pl.ds(j * IDX_PER_GATHER, IDX_PER_GATHER)],
                gsem,
            )
            for j in range(GATHERS)
        ]
        for cp in copies:
            cp.wait()

        def tok_body(t, tcarry):
            base = t * SEQ
            acc0 = jnp.zeros((HALF,), jnp.float32)
            acc1 = jnp.zeros((HALF,), jnp.float32)
            for l in range(SEQ):
                acc0 = acc0 + f_v[l, pl.ds(0, HALF)] * rows_v[base + l, pl.ds(0, HALF)]
                acc1 = acc1 + f_v[l, pl.ds(HALF, HALF)] * rows_v[base + l, pl.ds(HALF, HALF)]
            out_v[t, pl.ds(0, HALF)] = acc0
            out_v[t, pl.ds(HALF, HALF)] = acc1
            return tcarry

        lax.fori_loop(0, CHUNK_TOK, tok_body, 0)
        tok0 = wid * TOK_PER_W + c * CHUNK_TOK
        pltpu.sync_copy(out_v, out.at[pl.ds(tok0, CHUNK_TOK)])
        return carry

    lax.fori_loop(0, CHUNKS, chunk_body, 0)


@jax.jit
def kernel(input_sequence, embedding_table, f):
    B, N, L = input_sequence.shape
    idx1d = input_sequence.reshape(-1).astype(jnp.int32)
    mesh = plsc.VectorSubcoreMesh(core_axis_name="c", subcore_axis_name="s")
    out = pl.kernel(
        _body,
        out_type=jax.ShapeDtypeStruct((TOKENS, EMBED), jnp.float32),
        mesh=mesh,
        scratch_types=[
            pltpu.VMEM((ROWS_PER_CHUNK,), jnp.int32),
            pltpu.VMEM((ROWS_PER_CHUNK, EMBED), jnp.float32),
            pltpu.VMEM((SEQ, EMBED), jnp.float32),
            pltpu.VMEM((CHUNK_TOK, EMBED), jnp.float32),
            pltpu.SemaphoreType.DMA,
        ],
        compiler_params=pltpu.CompilerParams(use_tc_tiling_on_sc=False),
    )(embedding_table, idx1d, f)
    return out.reshape(B, N, EMBED)


# speedup vs baseline: 5.4660x; 1.1295x over previous
import functools

import jax
import jax.numpy as jnp
from jax import lax
from jax.experimental import pallas as pl
from jax.experimental.pallas import tpu as pltpu
from jax.experimental.pallas import tpu_sc as plsc

EMBED = 32
SEQ = 20
HALF = 16

NUM_WORKERS = 32
TOKENS = 4096 * 26
TOK_PER_W = TOKENS // NUM_WORKERS
CHUNK_TOK = 64
CHUNKS = TOK_PER_W // CHUNK_TOK
PAIRS = CHUNKS // 2
ROWS_PER_CHUNK = CHUNK_TOK * SEQ
IDX_PER_GATHER = 128
GATHERS = ROWS_PER_CHUNK // IDX_PER_GATHER


def _fire(table, idx, idx_v, rows_v, sem, wid, c):
    i0 = (wid * TOK_PER_W + c * CHUNK_TOK) * SEQ
    pltpu.sync_copy(idx.at[pl.ds(i0, ROWS_PER_CHUNK)], idx_v)
    for j in range(GATHERS):
        pltpu.async_copy(
            table.at[idx_v.at[pl.ds(j * IDX_PER_GATHER, IDX_PER_GATHER)]],
            rows_v.at[pl.ds(j * IDX_PER_GATHER, IDX_PER_GATHER)],
            sem,
        )


def _drain(table, idx_v, rows_v, sem):
    for j in range(GATHERS):
        pltpu.make_async_copy(
            table.at[idx_v.at[pl.ds(j * IDX_PER_GATHER, IDX_PER_GATHER)]],
            rows_v.at[pl.ds(j * IDX_PER_GATHER, IDX_PER_GATHER)],
            sem,
        ).wait()


def _compute(f_v, rows_v, out_v, out, wid, c):
    def tok_body(t, tcarry):
        base = t * SEQ
        acc0 = jnp.zeros((HALF,), jnp.float32)
        acc1 = jnp.zeros((HALF,), jnp.float32)
        for l in range(SEQ):
            acc0 = acc0 + f_v[l, pl.ds(0, HALF)] * rows_v[base + l, pl.ds(0, HALF)]
            acc1 = acc1 + f_v[l, pl.ds(HALF, HALF)] * rows_v[base + l, pl.ds(HALF, HALF)]
        out_v[t, pl.ds(0, HALF)] = acc0
        out_v[t, pl.ds(HALF, HALF)] = acc1
        return tcarry

    lax.fori_loop(0, CHUNK_TOK, tok_body, 0)
    tok0 = wid * TOK_PER_W + c * CHUNK_TOK
    pltpu.sync_copy(out_v, out.at[pl.ds(tok0, CHUNK_TOK)])


def _body(table, idx, f, out, idx_v, rows_v, f_v, out_v, sem0, sem1):
    wid = lax.axis_index("s") * 2 + lax.axis_index("c")
    pltpu.sync_copy(f, f_v)

    _fire(table, idx, idx_v.at[0], rows_v.at[0], sem0, wid, 0)

    def pair_body(g, carry):
        c0 = g * 2
        _fire(table, idx, idx_v.at[1], rows_v.at[1], sem1, wid, c0 + 1)
        _drain(table, idx_v.at[0], rows_v.at[0], sem0)
        _compute(f_v, rows_v.at[0], out_v, out, wid, c0)

        @pl.when(g < PAIRS - 1)
        def _():
            _fire(table, idx, idx_v.at[0], rows_v.at[0], sem0, wid, c0 + 2)

        _drain(table, idx_v.at[1], rows_v.at[1], sem1)
        _compute(f_v, rows_v.at[1], out_v, out, wid, c0 + 1)
        return carry

    lax.fori_loop(0, PAIRS, pair_body, 0)


@jax.jit
def kernel(input_sequence, embedding_table, f):
    B, N, L = input_sequence.shape
    idx1d = input_sequence.reshape(-1).astype(jnp.int32)
    mesh = plsc.VectorSubcoreMesh(core_axis_name="c", subcore_axis_name="s")
    out = pl.kernel(
        _body,
        out_type=jax.ShapeDtypeStruct((TOKENS, EMBED), jnp.float32),
        mesh=mesh,
        scratch_types=[
            pltpu.VMEM((2, ROWS_PER_CHUNK), jnp.int32),
            pltpu.VMEM((2, ROWS_PER_CHUNK, EMBED), jnp.float32),
            pltpu.VMEM((SEQ, EMBED), jnp.float32),
            pltpu.VMEM((CHUNK_TOK, EMBED), jnp.float32),
            pltpu.SemaphoreType.DMA,
            pltpu.SemaphoreType.DMA,
        ],
        compiler_params=pltpu.CompilerParams(use_tc_tiling_on_sc=False),
    )(embedding_table, idx1d, f)
    return out.reshape(B, N, EMBED)


# speedup vs baseline: 5.5582x; 1.0169x over previous
import functools

import jax
import jax.numpy as jnp
from jax import lax
from jax.experimental import pallas as pl
from jax.experimental.pallas import tpu as pltpu
from jax.experimental.pallas import tpu_sc as plsc

EMBED = 32
SEQ = 20
HALF = 16

NUM_WORKERS = 32
TOKENS = 4096 * 26
TOK_PER_W = TOKENS // NUM_WORKERS
CHUNK_TOK = 64
CHUNKS = TOK_PER_W // CHUNK_TOK
PAIRS = CHUNKS // 2
ROWS_PER_CHUNK = CHUNK_TOK * SEQ
IDX_PER_GATHER = 128
GATHERS = ROWS_PER_CHUNK // IDX_PER_GATHER


B_DIM = 4096
N_DIM = 26
B_PER_W = B_DIM // NUM_WORKERS
GB = 16
GROUPS = B_PER_W // GB
PLANE = N_DIM * SEQ
FLAT_PER_G = GB * PLANE
FLAT_PER_W = B_PER_W * PLANE


def _compact_body(seq, outf, stage_v, comp_v):
    wid = lax.axis_index("s") * 2 + lax.axis_index("c")
    b0w = wid * B_PER_W
    lanes = lax.iota(jnp.int32, 16)

    def group_body(g, carry):
        pltpu.sync_copy(seq.at[pl.ds(b0w + g * GB, GB)], stage_v)
        for p in range(GB):
            for n in range(N_DIM):
                pos = p * PLANE + n * SEQ
                comp_v[pl.ds(pos, 16)] = stage_v[p, n, pl.ds(0, 16)]
                comp_v[pl.ds(pos + 4, 16)] = stage_v[p, n, pl.ds(4, 16)]
        pltpu.sync_copy(
            comp_v.at[pl.ds(0, FLAT_PER_G)],
            outf.at[pl.ds(wid * FLAT_PER_W + g * FLAT_PER_G, FLAT_PER_G)],
        )
        return carry

    lax.fori_loop(0, GROUPS, group_body, 0)


def _fire(table, idx, idx_v, rows_v, sem, wid, c):
    i0 = (wid * TOK_PER_W + c * CHUNK_TOK) * SEQ
    pltpu.sync_copy(idx.at[pl.ds(i0, ROWS_PER_CHUNK)], idx_v)
    for j in range(GATHERS):
        pltpu.async_copy(
            table.at[idx_v.at[pl.ds(j * IDX_PER_GATHER, IDX_PER_GATHER)]],
            rows_v.at[pl.ds(j * IDX_PER_GATHER, IDX_PER_GATHER)],
            sem,
        )


def _drain(table, idx_v, rows_v, sem):
    for j in range(GATHERS):
        pltpu.make_async_copy(
            table.at[idx_v.at[pl.ds(j * IDX_PER_GATHER, IDX_PER_GATHER)]],
            rows_v.at[pl.ds(j * IDX_PER_GATHER, IDX_PER_GATHER)],
            sem,
        ).wait()


def _compute(f_v, rows_v, out_v, out, wid, c):
    def tok_body(t, tcarry):
        base = t * SEQ
        acc0 = jnp.zeros((HALF,), jnp.float32)
        acc1 = jnp.zeros((HALF,), jnp.float32)
        for l in range(SEQ):
            acc0 = acc0 + f_v[l, pl.ds(0, HALF)] * rows_v[base + l, pl.ds(0, HALF)]
            acc1 = acc1 + f_v[l, pl.ds(HALF, HALF)] * rows_v[base + l, pl.ds(HALF, HALF)]
        out_v[t, pl.ds(0, HALF)] = acc0
        out_v[t, pl.ds(HALF, HALF)] = acc1
        return tcarry

    lax.fori_loop(0, CHUNK_TOK, tok_body, 0)
    tok0 = wid * TOK_PER_W + c * CHUNK_TOK
    pltpu.sync_copy(out_v, out.at[pl.ds(tok0, CHUNK_TOK)])


def _body(table, idx, f, out, idx_v, rows_v, f_v, out_v, sem0, sem1):
    wid = lax.axis_index("s") * 2 + lax.axis_index("c")
    pltpu.sync_copy(f, f_v)

    _fire(table, idx, idx_v.at[0], rows_v.at[0], sem0, wid, 0)

    def pair_body(g, carry):
        c0 = g * 2
        _fire(table, idx, idx_v.at[1], rows_v.at[1], sem1, wid, c0 + 1)
        _drain(table, idx_v.at[0], rows_v.at[0], sem0)
        _compute(f_v, rows_v.at[0], out_v, out, wid, c0)

        @pl.when(g < PAIRS - 1)
        def _():
            _fire(table, idx, idx_v.at[0], rows_v.at[0], sem0, wid, c0 + 2)

        _drain(table, idx_v.at[1], rows_v.at[1], sem1)
        _compute(f_v, rows_v.at[1], out_v, out, wid, c0 + 1)
        return carry

    lax.fori_loop(0, PAIRS, pair_body, 0)


@jax.jit
def kernel(input_sequence, embedding_table, f):
    B, N, L = input_sequence.shape
    mesh = plsc.VectorSubcoreMesh(core_axis_name="c", subcore_axis_name="s")
    idx1d = pl.kernel(
        _compact_body,
        out_type=jax.ShapeDtypeStruct((TOKENS * SEQ,), jnp.int32),
        mesh=mesh,
        scratch_types=[
            pltpu.VMEM((GB, N_DIM, SEQ), jnp.int32),
            pltpu.VMEM((FLAT_PER_G + 16,), jnp.int32),
        ],
    )(input_sequence)
    out = pl.kernel(
        _body,
        out_type=jax.ShapeDtypeStruct((TOKENS, EMBED), jnp.float32),
        mesh=mesh,
        scratch_types=[
            pltpu.VMEM((2, ROWS_PER_CHUNK), jnp.int32),
            pltpu.VMEM((2, ROWS_PER_CHUNK, EMBED), jnp.float32),
            pltpu.VMEM((SEQ, EMBED), jnp.float32),
            pltpu.VMEM((CHUNK_TOK, EMBED), jnp.float32),
            pltpu.SemaphoreType.DMA,
            pltpu.SemaphoreType.DMA,
        ],
        compiler_params=pltpu.CompilerParams(use_tc_tiling_on_sc=False),
    )(embedding_table, idx1d, f)
    return out.reshape(B, N, EMBED)
